# baseline (speedup 1.0000x reference)
import functools

import jax
import jax.numpy as jnp
from jax import lax
from jax.experimental import pallas as pl
from jax.experimental.pallas import tpu as pltpu
from jax.experimental.pallas import tpu_sc as plsc

VOCAB = 400001
EMBED_DIM = 200
PAD_DIM = 256
BATCH = 4096
SEQ = 20
NUM_IDX = BATCH * SEQ
NUM_WORKERS = 32
ROWS_PER_WORKER = NUM_IDX // NUM_WORKERS
CHUNK = 128
NUM_CHUNKS = ROWS_PER_WORKER // CHUNK

_mesh = plsc.VectorSubcoreMesh(core_axis_name="c", subcore_axis_name="s")


@functools.partial(
    pl.kernel,
    mesh=_mesh,
    out_type=jax.ShapeDtypeStruct((NUM_IDX, PAD_DIM), jnp.float32),
    compiler_params=pltpu.CompilerParams(use_tc_tiling_on_sc=True),
    scratch_types=[
        pltpu.VMEM((ROWS_PER_WORKER,), jnp.int32),
        pltpu.VMEM((CHUNK, PAD_DIM), jnp.float32),
        pltpu.VMEM((CHUNK, PAD_DIM), jnp.float32),
        pltpu.SemaphoreType.DMA,
        pltpu.SemaphoreType.DMA,
    ],
)
def _embed_gather(idx_hbm, table_hbm, out_hbm, idx_v, buf0, buf1, sem0, sem1):
    wid = lax.axis_index("s") * 2 + lax.axis_index("c")
    base = wid * ROWS_PER_WORKER
    pltpu.sync_copy(idx_hbm.at[pl.ds(base, ROWS_PER_WORKER)], idx_v)

    bufs = (buf0, buf1)
    sems = (sem0, sem1)

    def start(j):
        isl = idx_v.at[pl.ds(j * CHUNK, CHUNK)]
        return pltpu.async_copy(table_hbm.at[isl], bufs[j % 2], sems[j % 2])

    copies = [None, None]
    copies[0] = start(0)
    for j in range(NUM_CHUNKS):
        if j + 1 < NUM_CHUNKS:
            copies[(j + 1) % 2] = start(j + 1)
        copies[j % 2].wait()
        pltpu.sync_copy(bufs[j % 2],
                        out_hbm.at[pl.ds(base + j * CHUNK, CHUNK)])


def kernel(x, table):
    idx = x.reshape(NUM_IDX).astype(jnp.int32)
    tpad = jnp.pad(table, ((0, 0), (0, PAD_DIM - EMBED_DIM)))
    out = _embed_gather(idx, tpad)
    return out[:, :EMBED_DIM].reshape(BATCH, SEQ, EMBED_DIM)


# speedup vs baseline: 4.0756x; 4.0756x over previous
import functools

import jax
import jax.numpy as jnp
from jax import lax
from jax.experimental import pallas as pl
from jax.experimental.pallas import tpu as pltpu
from jax.experimental.pallas import tpu_sc as plsc

VOCAB = 400001
EMBED_DIM = 200
PAD_DIM = 256
BATCH = 4096
SEQ = 20
NUM_IDX = BATCH * SEQ
NUM_WORKERS = 32
ROWS_PER_WORKER = NUM_IDX // NUM_WORKERS
CHUNK = 128
NUM_CHUNKS = ROWS_PER_WORKER // CHUNK

VB = 2048
NBLK = (VOCAB + VB - 1) // VB

_mesh = plsc.VectorSubcoreMesh(core_axis_name="c", subcore_axis_name="s")


def _tr_body(in_ref, out_ref):
    t = in_ref[...].T
    out_ref[...] = jnp.concatenate(
        [t, jnp.zeros((VB, PAD_DIM - EMBED_DIM), jnp.float32)], axis=1)


_transpose_pad = pl.pallas_call(
    _tr_body,
    grid=(NBLK,),
    in_specs=[pl.BlockSpec((EMBED_DIM, VB), lambda j: (0, j))],
    out_specs=pl.BlockSpec((VB, PAD_DIM), lambda j: (j, 0)),
    out_shape=jax.ShapeDtypeStruct((VOCAB, PAD_DIM), jnp.float32),
)


@functools.partial(
    pl.kernel,
    mesh=_mesh,
    out_type=jax.ShapeDtypeStruct((NUM_IDX, PAD_DIM), jnp.float32),
    compiler_params=pltpu.CompilerParams(use_tc_tiling_on_sc=True),
    scratch_types=[
        pltpu.VMEM((ROWS_PER_WORKER,), jnp.int32),
        pltpu.VMEM((CHUNK, PAD_DIM), jnp.float32),
        pltpu.VMEM((CHUNK, PAD_DIM), jnp.float32),
        pltpu.SemaphoreType.DMA,
        pltpu.SemaphoreType.DMA,
    ],
)
def _embed_gather(idx_hbm, table_hbm, out_hbm, idx_v, buf0, buf1, sem0, sem1):
    wid = lax.axis_index("s") * 2 + lax.axis_index("c")
    base = wid * ROWS_PER_WORKER
    pltpu.sync_copy(idx_hbm.at[pl.ds(base, ROWS_PER_WORKER)], idx_v)

    bufs = (buf0, buf1)
    sems = (sem0, sem1)

    def start(j):
        isl = idx_v.at[pl.ds(j * CHUNK, CHUNK)]
        return pltpu.async_copy(table_hbm.at[isl], bufs[j % 2], sems[j % 2])

    copies = [None, None]
    copies[0] = start(0)
    for j in range(NUM_CHUNKS):
        if j + 1 < NUM_CHUNKS:
            copies[(j + 1) % 2] = start(j + 1)
        copies[j % 2].wait()
        pltpu.sync_copy(bufs[j % 2],
                        out_hbm.at[pl.ds(base + j * CHUNK, CHUNK)])


def kernel(x, table):
    idx = x.reshape(NUM_IDX).astype(jnp.int32)
    tpad = _transpose_pad(table.T)
    out = _embed_gather(idx, tpad)
    return out[:, :EMBED_DIM].reshape(BATCH, SEQ, EMBED_DIM)


# speedup vs baseline: 4.5276x; 1.1109x over previous
import functools

import jax
import jax.numpy as jnp
from jax import lax
from jax.experimental import pallas as pl
from jax.experimental.pallas import tpu as pltpu
from jax.experimental.pallas import tpu_sc as plsc

VOCAB = 400001
EMBED_DIM = 200
PAD_DIM = 256
BATCH = 4096
SEQ = 20
NUM_IDX = BATCH * SEQ
NUM_WORKERS = 32
ROWS_PER_WORKER = NUM_IDX // NUM_WORKERS
CHUNK = 128
NUM_CHUNKS = ROWS_PER_WORKER // CHUNK

VB = 4096
NBLK = (VOCAB + VB - 1) // VB

_mesh = plsc.VectorSubcoreMesh(core_axis_name="c", subcore_axis_name="s")


def _tr_body(in_ref, out_ref):
    t = in_ref[...].T
    out_ref[...] = jnp.concatenate(
        [t, jnp.zeros((VB, PAD_DIM - EMBED_DIM), jnp.float32)], axis=1)


_transpose_pad = pl.pallas_call(
    _tr_body,
    grid=(NBLK,),
    in_specs=[pl.BlockSpec((EMBED_DIM, VB), lambda j: (0, j))],
    out_specs=pl.BlockSpec((VB, PAD_DIM), lambda j: (j, 0)),
    out_shape=jax.ShapeDtypeStruct((VOCAB, PAD_DIM), jnp.float32),
)


@functools.partial(
    pl.kernel,
    mesh=_mesh,
    out_type=jax.ShapeDtypeStruct((NUM_IDX, PAD_DIM), jnp.float32),
    compiler_params=pltpu.CompilerParams(use_tc_tiling_on_sc=True),
    scratch_types=[
        pltpu.VMEM((ROWS_PER_WORKER,), jnp.int32),
        pltpu.VMEM((CHUNK, PAD_DIM), jnp.float32),
        pltpu.VMEM((CHUNK, PAD_DIM), jnp.float32),
        pltpu.SemaphoreType.DMA,
        pltpu.SemaphoreType.DMA,
    ],
)
def _embed_gather(idx_hbm, table_hbm, out_hbm, idx_v, buf0, buf1, sem0, sem1):
    wid = lax.axis_index("s") * 2 + lax.axis_index("c")
    base = wid * ROWS_PER_WORKER
    pltpu.sync_copy(idx_hbm.at[pl.ds(base, ROWS_PER_WORKER)], idx_v)

    bufs = (buf0, buf1)
    sems = (sem0, sem1)

    def start(j):
        isl = idx_v.at[pl.ds(j * CHUNK, CHUNK)]
        return pltpu.async_copy(table_hbm.at[isl], bufs[j % 2], sems[j % 2])

    copies = [None, None]
    copies[0] = start(0)
    for j in range(NUM_CHUNKS):
        if j + 1 < NUM_CHUNKS:
            copies[(j + 1) % 2] = start(j + 1)
        copies[j % 2].wait()
        pltpu.sync_copy(bufs[j % 2],
                        out_hbm.at[pl.ds(base + j * CHUNK, CHUNK)])


def kernel(x, table):
    idx = x.reshape(NUM_IDX).astype(jnp.int32)
    tpad = _transpose_pad(table.T)
    out = _embed_gather(idx, tpad)
    return out[:, :EMBED_DIM].reshape(BATCH, SEQ, EMBED_DIM)


# speedup vs baseline: 4.6018x; 1.0164x over previous
import functools

import jax
import jax.numpy as jnp
from jax import lax
from jax.experimental import pallas as pl
from jax.experimental.pallas import tpu as pltpu
from jax.experimental.pallas import tpu_sc as plsc

VOCAB = 400001
EMBED_DIM = 200
PAD_DIM = 256
BATCH = 4096
SEQ = 20
NUM_IDX = BATCH * SEQ
NUM_WORKERS = 32
ROWS_PER_WORKER = NUM_IDX // NUM_WORKERS
CHUNK = 128
NUM_CHUNKS = ROWS_PER_WORKER // CHUNK

VB = 8192
NBLK = (VOCAB + VB - 1) // VB

_mesh = plsc.VectorSubcoreMesh(core_axis_name="c", subcore_axis_name="s")


def _tr_body(in_ref, out_ref):
    t = in_ref[...].T
    out_ref[...] = jnp.concatenate(
        [t, jnp.zeros((VB, PAD_DIM - EMBED_DIM), jnp.float32)], axis=1)


_transpose_pad = pl.pallas_call(
    _tr_body,
    grid=(NBLK,),
    in_specs=[pl.BlockSpec((EMBED_DIM, VB), lambda j: (0, j))],
    out_specs=pl.BlockSpec((VB, PAD_DIM), lambda j: (j, 0)),
    out_shape=jax.ShapeDtypeStruct((VOCAB, PAD_DIM), jnp.float32),
)


@functools.partial(
    pl.kernel,
    mesh=_mesh,
    out_type=jax.ShapeDtypeStruct((NUM_IDX, PAD_DIM), jnp.float32),
    compiler_params=pltpu.CompilerParams(use_tc_tiling_on_sc=True),
    scratch_types=[
        pltpu.VMEM((ROWS_PER_WORKER,), jnp.int32),
        pltpu.VMEM((CHUNK, PAD_DIM), jnp.float32),
        pltpu.VMEM((CHUNK, PAD_DIM), jnp.float32),
        pltpu.SemaphoreType.DMA,
        pltpu.SemaphoreType.DMA,
    ],
)
def _embed_gather(idx_hbm, table_hbm, out_hbm, idx_v, buf0, buf1, sem0, sem1):
    wid = lax.axis_index("s") * 2 + lax.axis_index("c")
    base = wid * ROWS_PER_WORKER
    pltpu.sync_copy(idx_hbm.at[pl.ds(base, ROWS_PER_WORKER)], idx_v)

    bufs = (buf0, buf1)
    sems = (sem0, sem1)

    def start(j):
        isl = idx_v.at[pl.ds(j * CHUNK, CHUNK)]
        return pltpu.async_copy(table_hbm.at[isl], bufs[j % 2], sems[j % 2])

    copies = [None, None]
    copies[0] = start(0)
    for j in range(NUM_CHUNKS):
        if j + 1 < NUM_CHUNKS:
            copies[(j + 1) % 2] = start(j + 1)
        copies[j % 2].wait()
        pltpu.sync_copy(bufs[j % 2],
                        out_hbm.at[pl.ds(base + j * CHUNK, CHUNK)])


def kernel(x, table):
    idx = x.reshape(NUM_IDX).astype(jnp.int32)
    tpad = _transpose_pad(table.T)
    out = _embed_gather(idx, tpad)
    return out[:, :EMBED_DIM].reshape(BATCH, SEQ, EMBED_DIM)


# speedup vs baseline: 4.6101x; 1.0018x over previous
import functools

import jax
import jax.numpy as jnp
from jax import lax
from jax.experimental import pallas as pl
from jax.experimental.pallas import tpu as pltpu
from jax.experimental.pallas import tpu_sc as plsc

VOCAB = 400001
EMBED_DIM = 200
PAD_DIM = 256
BATCH = 4096
SEQ = 20
NUM_IDX = BATCH * SEQ
NUM_WORKERS = 32
ROWS_PER_WORKER = NUM_IDX // NUM_WORKERS
CHUNK = 128
NUM_CHUNKS = ROWS_PER_WORKER // CHUNK

VB = 8192
NBLK = (VOCAB + VB - 1) // VB

_mesh = plsc.VectorSubcoreMesh(core_axis_name="c", subcore_axis_name="s")


def _tr_body(in_ref, out_ref):
    t = in_ref[...].T
    out_ref[...] = jnp.concatenate(
        [t, jnp.zeros((VB, PAD_DIM - EMBED_DIM), jnp.float32)], axis=1)


_transpose_pad = pl.pallas_call(
    _tr_body,
    grid=(NBLK,),
    in_specs=[pl.BlockSpec((EMBED_DIM, VB), lambda j: (0, j))],
    out_specs=pl.BlockSpec((VB, PAD_DIM), lambda j: (j, 0)),
    out_shape=jax.ShapeDtypeStruct((VOCAB, PAD_DIM), jnp.float32),
)


@functools.partial(
    pl.kernel,
    mesh=_mesh,
    out_type=jax.ShapeDtypeStruct((NUM_IDX, PAD_DIM), jnp.float32),
    compiler_params=pltpu.CompilerParams(use_tc_tiling_on_sc=True),
    scratch_types=[
        pltpu.VMEM((ROWS_PER_WORKER,), jnp.int32),
        pltpu.VMEM((CHUNK, PAD_DIM), jnp.float32),
        pltpu.VMEM((CHUNK, PAD_DIM), jnp.float32),
        pltpu.SemaphoreType.DMA,
        pltpu.SemaphoreType.DMA,
    ],
)
def _embed_gather(idx_hbm, table_hbm, out_hbm, idx_v, buf0, buf1, sem0, sem1):
    wid = lax.axis_index("s") * 2 + lax.axis_index("c")
    base = wid * ROWS_PER_WORKER
    pltpu.sync_copy(idx_hbm.at[pl.ds(base, ROWS_PER_WORKER)], idx_v)

    bufs = (buf0, buf1)
    sems = (sem0, sem1)

    def start(j):
        isl = idx_v.at[pl.ds(j * CHUNK, CHUNK)]
        return pltpu.async_copy(table_hbm.at[isl], bufs[j % 2], sems[j % 2])

    copies = [None, None]
    copies[0] = start(0)
    for j in range(NUM_CHUNKS):
        if j + 1 < NUM_CHUNKS:
            copies[(j + 1) % 2] = start(j + 1)
        copies[j % 2].wait()
        pltpu.sync_copy(bufs[j % 2],
                        out_hbm.at[pl.ds(base + j * CHUNK, CHUNK)])


def kernel(x, table):
    idx = x.reshape(NUM_IDX).astype(jnp.int32)
    tpad = _transpose_pad(table.T)
    out = _embed_gather(idx, tpad)
    return out.reshape(BATCH, SEQ, PAD_DIM)[:, :, :EMBED_DIM]


# speedup vs baseline: 4.6468x; 1.0080x over previous
import functools

import jax
import jax.numpy as jnp
from jax import lax
from jax.experimental import pallas as pl
from jax.experimental.pallas import tpu as pltpu
from jax.experimental.pallas import tpu_sc as plsc

VOCAB = 400001
EMBED_DIM = 200
PAD_DIM = 256
BATCH = 4096
SEQ = 20
NUM_IDX = BATCH * SEQ
NUM_WORKERS = 32
ROWS_PER_WORKER = NUM_IDX // NUM_WORKERS
CHUNK = 128
NUM_CHUNKS = ROWS_PER_WORKER // CHUNK

VB = 12288
NBLK = (VOCAB + VB - 1) // VB

_mesh = plsc.VectorSubcoreMesh(core_axis_name="c", subcore_axis_name="s")


def _tr_body(in_ref, out_ref):
    t = in_ref[...].T
    out_ref[...] = jnp.concatenate(
        [t, jnp.zeros((VB, PAD_DIM - EMBED_DIM), jnp.float32)], axis=1)


_transpose_pad = pl.pallas_call(
    _tr_body,
    grid=(NBLK,),
    in_specs=[pl.BlockSpec((EMBED_DIM, VB), lambda j: (0, j))],
    out_specs=pl.BlockSpec((VB, PAD_DIM), lambda j: (j, 0)),
    out_shape=jax.ShapeDtypeStruct((VOCAB, PAD_DIM), jnp.float32),
)


@functools.partial(
    pl.kernel,
    mesh=_mesh,
    out_type=jax.ShapeDtypeStruct((NUM_IDX, PAD_DIM), jnp.float32),
    compiler_params=pltpu.CompilerParams(use_tc_tiling_on_sc=True),
    scratch_types=[
        pltpu.VMEM((ROWS_PER_WORKER,), jnp.int32),
        pltpu.VMEM((CHUNK, PAD_DIM), jnp.float32),
        pltpu.VMEM((CHUNK, PAD_DIM), jnp.float32),
        pltpu.VMEM((CHUNK, PAD_DIM), jnp.float32),
        pltpu.SemaphoreType.DMA,
        pltpu.SemaphoreType.DMA,
        pltpu.SemaphoreType.DMA,
    ],
)
def _embed_gather(idx_hbm, table_hbm, out_hbm, idx_v,
                  buf0, buf1, buf2, sem0, sem1, sem2):
    wid = lax.axis_index("s") * 2 + lax.axis_index("c")
    base = wid * ROWS_PER_WORKER
    pltpu.sync_copy(idx_hbm.at[pl.ds(base, ROWS_PER_WORKER)], idx_v)

    bufs = (buf0, buf1, buf2)
    sems = (sem0, sem1, sem2)
    nbuf = len(bufs)

    def start(j):
        isl = idx_v.at[pl.ds(j * CHUNK, CHUNK)]
        return pltpu.async_copy(table_hbm.at[isl], bufs[j % nbuf], sems[j % nbuf])

    copies = [None] * nbuf
    for j in range(nbuf - 1):
        copies[j] = start(j)
    for j in range(NUM_CHUNKS):
        if j + nbuf - 1 < NUM_CHUNKS:
            copies[(j + nbuf - 1) % nbuf] = start(j + nbuf - 1)
        copies[j % nbuf].wait()
        pltpu.sync_copy(bufs[j % nbuf],
                        out_hbm.at[pl.ds(base + j * CHUNK, CHUNK)])


def kernel(x, table):
    idx = x.reshape(NUM_IDX).astype(jnp.int32)
    tpad = _transpose_pad(table.T)
    out = _embed_gather(idx, tpad)
    return out.reshape(BATCH, SEQ, PAD_DIM)[:, :, :EMBED_DIM]
